# baseline (speedup 1.0000x reference)
import functools

import jax
import jax.numpy as jnp
from jax import lax
from jax.experimental import pallas as pl
from jax.experimental.pallas import tpu as pltpu
from jax.experimental.pallas import tpu_sc as plsc

_NC = 2
_NS = 16
_NW = _NC * _NS

_CHUNK = 128


@functools.lru_cache(maxsize=None)
def _build(V, D, B, C):
    b_per_w = B // _NW
    n_chunks = b_per_w // C
    mesh = plsc.VectorSubcoreMesh(core_axis_name="c", subcore_axis_name="s")

    @functools.partial(
        pl.kernel,
        out_type=jax.ShapeDtypeStruct((B, D), jnp.float32),
        mesh=mesh,
        compiler_params=pltpu.CompilerParams(use_tc_tiling_on_sc=False),
        scratch_types=[
            pltpu.VMEM((n_chunks, C), jnp.int32),
            pltpu.VMEM((C, D), jnp.float32),
            pltpu.SemaphoreType.DMA,
        ],
    )
    def k(xs_hbm, emb_hbm, out_hbm, idx_v, rows, gsem):
        wid = lax.axis_index("s") * _NC + lax.axis_index("c")
        base = wid * b_per_w
        pltpu.sync_copy(xs_hbm.at[wid], idx_v)

        def body(c, carry):
            pltpu.make_async_copy(emb_hbm.at[idx_v.at[c]], rows, gsem).start()
            pltpu.make_async_copy(emb_hbm.at[idx_v.at[c]], rows, gsem).wait()
            pltpu.sync_copy(rows, out_hbm.at[pl.ds(base + c * C, C)])
            return carry

        lax.fori_loop(0, n_chunks, body, 0)

    return k


def kernel(xs, emb):
    Bdim, S = xs.shape
    V, D = emb.shape
    B = Bdim * S
    C = _CHUNK
    assert B % (_NW * C) == 0
    n_chunks = B // (_NW * C)
    xs_flat = xs.reshape(_NW, n_chunks, C)
    out = _build(V, D, B, C)(xs_flat, emb)
    return out.reshape(Bdim, S, D)
